# baseline (speedup 1.0000x reference)
import functools
import math

import jax
import jax.numpy as jnp
from jax import lax
from jax.experimental import pallas as pl
from jax.experimental.pallas import tpu as pltpu
from jax.experimental.pallas import tpu_sc as plsc

CAPACITY_FACTOR = 0.5


def _route_body(k, scores_ref, pos_ref, aux_ref):
    s = scores_ref[...]
    B, T = s.shape
    ku = T - k
    si = lax.bitcast_convert_type(s, jnp.int32)
    u = jnp.where(si < 0, ~si, si | jnp.int32(-2147483648))
    u = lax.bitcast_convert_type(u, jnp.uint32)

    def tbody(i, t):
        bit = jnp.uint32(1) << (jnp.uint32(31) - i.astype(jnp.uint32))
        cand = t | bit
        cnt = jnp.sum((u >= cand).astype(jnp.int32), axis=1, keepdims=True)
        return jnp.where(cnt >= k, cand, t)

    t = lax.fori_loop(0, 32, tbody, jnp.zeros((B, 1), jnp.uint32))

    cnt_gt = jnp.sum((u > t).astype(jnp.int32), axis=1, keepdims=True)
    need = k - cnt_gt
    ties = u == t
    iot = lax.broadcasted_iota(jnp.int32, (B, T), 1)

    def cbody(i, c):
        cand = c | (jnp.int32(1) << (jnp.int32(12) - i))
        cntc = jnp.sum((ties & (iot < cand)).astype(jnp.int32), axis=1,
                       keepdims=True)
        return jnp.where(cntc <= need, cand, c)

    c = lax.fori_loop(0, 13, cbody, jnp.zeros((B, 1), jnp.int32))

    mask = (u > t) | (ties & (iot < c))

    L = 128
    NR = (B * T) // L
    RPB = T // L
    m2 = mask.astype(jnp.float32).reshape(NR, L)
    r = lax.broadcasted_iota(jnp.int32, (L, L), 0)
    cc = lax.broadcasted_iota(jnp.int32, (L, L), 1)
    tri_incl = (r <= cc).astype(jnp.float32)
    c2 = jnp.dot(m2, tri_incl, preferred_element_type=jnp.float32)
    same = (r // RPB) == (cc // RPB)
    strict = cc < r
    m_pref = (same & strict).astype(jnp.float32)
    rowsum = c2[:, L - 1:L]
    base = jnp.dot(m_pref, rowsum, preferred_element_type=jnp.float32)
    csum2 = (c2 + base).astype(jnp.int32)
    mask2 = m2 == 1.0
    rid = lax.broadcasted_iota(jnp.int32, (NR, L), 0)
    lid = lax.broadcasted_iota(jnp.int32, (NR, L), 1)
    brow2 = rid // RPB
    iot2 = (rid % RPB) * L + lid
    pos2 = jnp.where(mask2,
                     brow2 * k + (csum2 - 1),
                     B * k + brow2 * ku + (iot2 - csum2))
    pos_ref[...] = pos2

    p = jax.nn.sigmoid(s)
    mp = jnp.mean(p, axis=1)
    aux_ref[...] = jnp.mean((mp - CAPACITY_FACTOR) ** 2).reshape(1, 1)


def _route(scores, k):
    B, T = scores.shape
    assert (B * T) % 128 == 0 and (B * T) // 128 == 128
    return pl.pallas_call(
        functools.partial(_route_body, k),
        out_shape=[
            jax.ShapeDtypeStruct(((B * T) // 128, 128), jnp.int32),
            jax.ShapeDtypeStruct((1, 1), jnp.float32),
        ],
    )(scores)


def _invert(pos_flat, tok_ids):
    (N,) = pos_flat.shape
    info = plsc.get_sparse_core_info()
    NC, NS = info.num_cores, info.num_subcores
    NW = NC * NS
    R = N // NW
    CH = 128
    mesh = plsc.VectorSubcoreMesh(core_axis_name="c", subcore_axis_name="s")

    @functools.partial(
        pl.kernel,
        out_type=jax.ShapeDtypeStruct((N,), jnp.int32),
        mesh=mesh,
        scratch_types=[
            pltpu.VMEM((CH,), jnp.int32),
            pltpu.VMEM((CH,), jnp.int32),
            pltpu.SemaphoreType.DMA,
        ],
    )
    def invert(pos_hbm, tok_hbm, inv_hbm, pos_v, tok_v, sem):
        wid = lax.axis_index("s") * NC + lax.axis_index("c")
        base = wid * R

        def body(i, _):
            o = base + i * CH
            pltpu.sync_copy(pos_hbm.at[pl.ds(o, CH)], pos_v)
            pltpu.sync_copy(tok_hbm.at[pl.ds(o, CH)], tok_v)
            pltpu.async_copy(tok_v, inv_hbm.at[pos_v], sem).wait()
            return 0

        lax.fori_loop(0, R // CH, body, 0)

    return invert(pos_flat, tok_ids)


def _gather(x2d, inv, nsel, D, idx_base):
    info = plsc.get_sparse_core_info()
    NC, NS = info.num_cores, info.num_subcores
    NW = NC * NS
    R = nsel // NW
    CH = 32
    mesh = plsc.VectorSubcoreMesh(core_axis_name="c", subcore_axis_name="s")

    @functools.partial(
        pl.kernel,
        out_type=jax.ShapeDtypeStruct((nsel, D), jnp.float32),
        mesh=mesh,
        scratch_types=[
            pltpu.VMEM((CH,), jnp.int32),
            pltpu.VMEM((CH, D), jnp.float32),
            pltpu.SemaphoreType.DMA,
        ],
    )
    def gather(x_hbm, idx_hbm, out_hbm, idx_v, rows_v, sem):
        wid = lax.axis_index("s") * NC + lax.axis_index("c")
        base = wid * R

        def body(i, _):
            o = base + i * CH
            pltpu.sync_copy(idx_hbm.at[pl.ds(idx_base + o, CH)], idx_v)
            pltpu.async_copy(x_hbm.at[idx_v], rows_v, sem).wait()
            pltpu.sync_copy(rows_v, out_hbm.at[pl.ds(o, CH)])
            return 0

        lax.fori_loop(0, R // CH, body, 0)

    return gather(x2d, inv)


def _ffn_body(x_ref, w1_ref, b1_ref, w2_ref, b2_ref, y_ref):
    f = pl.program_id(1)

    @pl.when(f == 0)
    def _():
        y_ref[...] = jnp.broadcast_to(b2_ref[...], y_ref.shape)

    h = jnp.dot(x_ref[...].astype(jnp.bfloat16), w1_ref[...],
                preferred_element_type=jnp.float32)
    h = jax.nn.gelu(h + b1_ref[...])
    y_ref[...] += jnp.dot(h.astype(jnp.bfloat16), w2_ref[...],
                          preferred_element_type=jnp.float32)


def _ffn(xc, W1, b1, W2, b2, BM=512, BF=2048):
    M, D = xc.shape
    _, F = W1.shape
    grid = (M // BM, F // BF)
    return pl.pallas_call(
        _ffn_body,
        grid=grid,
        in_specs=[
            pl.BlockSpec((BM, D), lambda m, f: (m, 0)),
            pl.BlockSpec((D, BF), lambda m, f: (0, f)),
            pl.BlockSpec((1, BF), lambda m, f: (0, f)),
            pl.BlockSpec((BF, D), lambda m, f: (f, 0)),
            pl.BlockSpec((1, D), lambda m, f: (0, 0)),
        ],
        out_specs=pl.BlockSpec((BM, D), lambda m, f: (m, 0)),
        out_shape=jax.ShapeDtypeStruct((M, D), jnp.float32),
        compiler_params=pltpu.CompilerParams(
            dimension_semantics=("parallel", "arbitrary"),
        ),
    )(xc, W1.astype(jnp.bfloat16), b1.reshape(1, F),
      W2.astype(jnp.bfloat16), b2.reshape(1, D))


def _scatter(y_a, y_b, x2d, inv, D):
    (HSEL, _) = y_a.shape
    NSEL = 2 * HSEL
    (NTOT, _) = x2d.shape
    info = plsc.get_sparse_core_info()
    NC, NS = info.num_cores, info.num_subcores
    NW = NC * NS
    HW = NW // 2
    RS = NSEL // NW
    RU = (NTOT - NSEL) // NW
    CH = 32
    mesh = plsc.VectorSubcoreMesh(core_axis_name="c", subcore_axis_name="s")

    @functools.partial(
        pl.kernel,
        out_type=jax.ShapeDtypeStruct((NTOT, D), jnp.float32),
        mesh=mesh,
        scratch_types=[
            pltpu.VMEM((CH,), jnp.int32),
            pltpu.VMEM((CH, D), jnp.float32),
            pltpu.SemaphoreType.DMA,
        ],
    )
    def scatter(ya_hbm, yb_hbm, x_hbm, inv_hbm, out_hbm, idx_v, rows_v, sem):
        wid = lax.axis_index("s") * NC + lax.axis_index("c")

        def mk_body_sel(y_hbm, lbase):
            def body_sel(i, _):
                o = wid * RS + i * CH
                pltpu.sync_copy(inv_hbm.at[pl.ds(o, CH)], idx_v)
                pltpu.sync_copy(y_hbm.at[pl.ds(lbase + i * CH, CH)], rows_v)
                pltpu.async_copy(rows_v, out_hbm.at[idx_v], sem).wait()
                return 0
            return body_sel

        @pl.when(wid < HW)
        def _():
            lax.fori_loop(0, RS // CH, mk_body_sel(ya_hbm, wid * RS), 0)

        @pl.when(wid >= HW)
        def _():
            lax.fori_loop(0, RS // CH,
                          mk_body_sel(yb_hbm, (wid - HW) * RS), 0)

        def body_unsel(i, _):
            o = wid * RU + i * CH
            pltpu.sync_copy(inv_hbm.at[pl.ds(NSEL + o, CH)], idx_v)
            pltpu.async_copy(x_hbm.at[idx_v], rows_v, sem).wait()
            pltpu.async_copy(rows_v, out_hbm.at[idx_v], sem).wait()
            return 0

        lax.fori_loop(0, RU // CH, body_unsel, 0)

    return scatter(y_a, y_b, x2d, inv)


def kernel(x, w_gate, W1, b1, W2, b2):
    B, T, D = x.shape
    k = max(1, math.ceil(CAPACITY_FACTOR * T))
    scores = jnp.einsum('btd,d->bt', x, w_gate)
    pos, aux = _route(scores, k)
    tok_ids = jnp.arange(B * T, dtype=jnp.int32)
    inv = _invert(pos.reshape(B * T), tok_ids)
    x2d = x.reshape(B * T, D)
    half = (B * k) // 2
    xc_a = _gather(x2d, inv, half, D, 0)
    xc_b = _gather(x2d, inv, half, D, half)
    y_a = _ffn(xc_a, W1, b1, W2, b2)
    y_b = _ffn(xc_b, W1, b1, W2, b2)
    out2d = _scatter(y_a, y_b, x2d, inv, D)
    return out2d.reshape(B, T, D), aux[0, 0]


# speedup vs baseline: 1.0207x; 1.0207x over previous
import functools
import math

import jax
import jax.numpy as jnp
from jax import lax
from jax.experimental import pallas as pl
from jax.experimental.pallas import tpu as pltpu
from jax.experimental.pallas import tpu_sc as plsc

CAPACITY_FACTOR = 0.5


def _route_body(k, scores_ref, pos_ref, aux_ref):
    s = scores_ref[...]
    B, T = s.shape
    ku = T - k
    si = lax.bitcast_convert_type(s, jnp.int32)
    u = jnp.where(si < 0, ~si, si | jnp.int32(-2147483648))
    u = lax.bitcast_convert_type(u, jnp.uint32)

    def tbody(i, t):
        bit = jnp.uint32(1) << (jnp.uint32(31) - i.astype(jnp.uint32))
        cand = t | bit
        cnt = jnp.sum((u >= cand).astype(jnp.int32), axis=1, keepdims=True)
        return jnp.where(cnt >= k, cand, t)

    t = lax.fori_loop(0, 32, tbody, jnp.zeros((B, 1), jnp.uint32))

    cnt_gt = jnp.sum((u > t).astype(jnp.int32), axis=1, keepdims=True)
    need = k - cnt_gt
    ties = u == t
    iot = lax.broadcasted_iota(jnp.int32, (B, T), 1)

    def cbody(i, c):
        cand = c | (jnp.int32(1) << (jnp.int32(12) - i))
        cntc = jnp.sum((ties & (iot < cand)).astype(jnp.int32), axis=1,
                       keepdims=True)
        return jnp.where(cntc <= need, cand, c)

    c = lax.fori_loop(0, 13, cbody, jnp.zeros((B, 1), jnp.int32))

    mask = (u > t) | (ties & (iot < c))

    L = 128
    NR = (B * T) // L
    RPB = T // L
    m2 = mask.astype(jnp.float32).reshape(NR, L)
    r = lax.broadcasted_iota(jnp.int32, (L, L), 0)
    cc = lax.broadcasted_iota(jnp.int32, (L, L), 1)
    tri_incl = (r <= cc).astype(jnp.float32)
    c2 = jnp.dot(m2, tri_incl, preferred_element_type=jnp.float32)
    same = (r // RPB) == (cc // RPB)
    strict = cc < r
    m_pref = (same & strict).astype(jnp.float32)
    rowsum = c2[:, L - 1:L]
    base = jnp.dot(m_pref, rowsum, preferred_element_type=jnp.float32)
    csum2 = (c2 + base).astype(jnp.int32)
    mask2 = m2 == 1.0
    rid = lax.broadcasted_iota(jnp.int32, (NR, L), 0)
    lid = lax.broadcasted_iota(jnp.int32, (NR, L), 1)
    brow2 = rid // RPB
    iot2 = (rid % RPB) * L + lid
    pos2 = jnp.where(mask2,
                     brow2 * k + (csum2 - 1),
                     B * k + brow2 * ku + (iot2 - csum2))
    pos_ref[...] = pos2

    p = jax.nn.sigmoid(s)
    mp = jnp.mean(p, axis=1)
    aux_ref[...] = jnp.mean((mp - CAPACITY_FACTOR) ** 2).reshape(1, 1)


def _route(scores, k):
    B, T = scores.shape
    assert (B * T) % 128 == 0 and (B * T) // 128 == 128
    return pl.pallas_call(
        functools.partial(_route_body, k),
        out_shape=[
            jax.ShapeDtypeStruct(((B * T) // 128, 128), jnp.int32),
            jax.ShapeDtypeStruct((1, 1), jnp.float32),
        ],
    )(scores)


def _invert(pos_flat, tok_ids):
    (N,) = pos_flat.shape
    info = plsc.get_sparse_core_info()
    NC, NS = info.num_cores, info.num_subcores
    NW = NC * NS
    R = N // NW
    CH = 128
    mesh = plsc.VectorSubcoreMesh(core_axis_name="c", subcore_axis_name="s")

    @functools.partial(
        pl.kernel,
        out_type=jax.ShapeDtypeStruct((N,), jnp.int32),
        mesh=mesh,
        scratch_types=[
            pltpu.VMEM((CH,), jnp.int32),
            pltpu.VMEM((CH,), jnp.int32),
            pltpu.SemaphoreType.DMA,
        ],
    )
    def invert(pos_hbm, tok_hbm, inv_hbm, pos_v, tok_v, sem):
        wid = lax.axis_index("s") * NC + lax.axis_index("c")
        base = wid * R

        def body(i, _):
            o = base + i * CH
            pltpu.sync_copy(pos_hbm.at[pl.ds(o, CH)], pos_v)
            pltpu.sync_copy(tok_hbm.at[pl.ds(o, CH)], tok_v)
            pltpu.async_copy(tok_v, inv_hbm.at[pos_v], sem).wait()
            return 0

        lax.fori_loop(0, R // CH, body, 0)

    return invert(pos_flat, tok_ids)


def _gather(x2d, inv, nsel, D, idx_base):
    info = plsc.get_sparse_core_info()
    NC, NS = info.num_cores, info.num_subcores
    NW = NC * NS
    R = nsel // NW
    CH = 16
    mesh = plsc.VectorSubcoreMesh(core_axis_name="c", subcore_axis_name="s")

    @functools.partial(
        pl.kernel,
        out_type=jax.ShapeDtypeStruct((nsel, D), jnp.float32),
        mesh=mesh,
        scratch_types=[
            pltpu.VMEM((CH,), jnp.int32),
            pltpu.VMEM((CH, D), jnp.float32),
            pltpu.VMEM((CH,), jnp.int32),
            pltpu.VMEM((CH, D), jnp.float32),
            pltpu.SemaphoreType.DMA,
            pltpu.SemaphoreType.DMA,
        ],
    )
    def gather(x_hbm, idx_hbm, out_hbm, idx_a, rows_a, idx_b, rows_b,
               sem_g, sem_w):
        wid = lax.axis_index("s") * NC + lax.axis_index("c")
        base = wid * R

        def body(j, _):
            o0 = base + (2 * j) * CH
            o1 = o0 + CH
            pltpu.sync_copy(idx_hbm.at[pl.ds(idx_base + o0, CH)], idx_a)
            ga = pltpu.async_copy(x_hbm.at[idx_a], rows_a, sem_g)
            pltpu.sync_copy(idx_hbm.at[pl.ds(idx_base + o1, CH)], idx_b)
            gb = pltpu.async_copy(x_hbm.at[idx_b], rows_b, sem_g)
            ga.wait()
            wa = pltpu.async_copy(rows_a, out_hbm.at[pl.ds(o0, CH)], sem_w)
            gb.wait()
            wb = pltpu.async_copy(rows_b, out_hbm.at[pl.ds(o1, CH)], sem_w)
            wa.wait()
            wb.wait()
            return 0

        lax.fori_loop(0, R // (2 * CH), body, 0)

    return gather(x2d, inv)


def _ffn_body(x_ref, w1_ref, b1_ref, w2_ref, b2_ref, y_ref):
    f = pl.program_id(1)

    @pl.when(f == 0)
    def _():
        y_ref[...] = jnp.broadcast_to(b2_ref[...], y_ref.shape)

    h = jnp.dot(x_ref[...].astype(jnp.bfloat16), w1_ref[...],
                preferred_element_type=jnp.float32)
    h = jax.nn.gelu(h + b1_ref[...])
    y_ref[...] += jnp.dot(h.astype(jnp.bfloat16), w2_ref[...],
                          preferred_element_type=jnp.float32)


def _ffn(xc, W1, b1, W2, b2, BM=512, BF=2048):
    M, D = xc.shape
    _, F = W1.shape
    grid = (M // BM, F // BF)
    return pl.pallas_call(
        _ffn_body,
        grid=grid,
        in_specs=[
            pl.BlockSpec((BM, D), lambda m, f: (m, 0)),
            pl.BlockSpec((D, BF), lambda m, f: (0, f)),
            pl.BlockSpec((1, BF), lambda m, f: (0, f)),
            pl.BlockSpec((BF, D), lambda m, f: (f, 0)),
            pl.BlockSpec((1, D), lambda m, f: (0, 0)),
        ],
        out_specs=pl.BlockSpec((BM, D), lambda m, f: (m, 0)),
        out_shape=jax.ShapeDtypeStruct((M, D), jnp.float32),
        compiler_params=pltpu.CompilerParams(
            dimension_semantics=("parallel", "arbitrary"),
        ),
    )(xc, W1.astype(jnp.bfloat16), b1.reshape(1, F),
      W2.astype(jnp.bfloat16), b2.reshape(1, D))


def _scatter(y, x2d, inv, D):
    (NSEL, _) = y.shape
    (NTOT, _) = x2d.shape
    info = plsc.get_sparse_core_info()
    NC, NS = info.num_cores, info.num_subcores
    NW = NC * NS
    RS = NSEL // NW
    RU = (NTOT - NSEL) // NW
    CH = 16
    mesh = plsc.VectorSubcoreMesh(core_axis_name="c", subcore_axis_name="s")

    @functools.partial(
        pl.kernel,
        out_type=jax.ShapeDtypeStruct((NTOT, D), jnp.float32),
        mesh=mesh,
        scratch_types=[
            pltpu.VMEM((CH,), jnp.int32),
            pltpu.VMEM((CH, D), jnp.float32),
            pltpu.VMEM((CH,), jnp.int32),
            pltpu.VMEM((CH, D), jnp.float32),
            pltpu.SemaphoreType.DMA,
            pltpu.SemaphoreType.DMA,
        ],
    )
    def scatter(y_hbm, x_hbm, inv_hbm, out_hbm, idx_a, rows_a,
                idx_b, rows_b, sem_g, sem_w):
        wid = lax.axis_index("s") * NC + lax.axis_index("c")

        def body_sel(j, _):
            o0 = wid * RS + (2 * j) * CH
            o1 = o0 + CH
            pltpu.sync_copy(inv_hbm.at[pl.ds(o0, CH)], idx_a)
            ra = pltpu.async_copy(y_hbm.at[pl.ds(o0, CH)], rows_a, sem_g)
            pltpu.sync_copy(inv_hbm.at[pl.ds(o1, CH)], idx_b)
            rb = pltpu.async_copy(y_hbm.at[pl.ds(o1, CH)], rows_b, sem_g)
            ra.wait()
            wa = pltpu.async_copy(rows_a, out_hbm.at[idx_a], sem_w)
            rb.wait()
            wb = pltpu.async_copy(rows_b, out_hbm.at[idx_b], sem_w)
            wa.wait()
            wb.wait()
            return 0

        lax.fori_loop(0, RS // (2 * CH), body_sel, 0)

        def body_unsel(j, _):
            o0 = wid * RU + (2 * j) * CH
            o1 = o0 + CH
            pltpu.sync_copy(inv_hbm.at[pl.ds(NSEL + o0, CH)], idx_a)
            ga = pltpu.async_copy(x_hbm.at[idx_a], rows_a, sem_g)
            pltpu.sync_copy(inv_hbm.at[pl.ds(NSEL + o1, CH)], idx_b)
            gb = pltpu.async_copy(x_hbm.at[idx_b], rows_b, sem_g)
            ga.wait()
            wa = pltpu.async_copy(rows_a, out_hbm.at[idx_a], sem_w)
            gb.wait()
            wb = pltpu.async_copy(rows_b, out_hbm.at[idx_b], sem_w)
            wa.wait()
            wb.wait()
            return 0

        lax.fori_loop(0, RU // (2 * CH), body_unsel, 0)

    return scatter(y, x2d, inv)


def kernel(x, w_gate, W1, b1, W2, b2):
    B, T, D = x.shape
    k = max(1, math.ceil(CAPACITY_FACTOR * T))
    scores = jnp.einsum('btd,d->bt', x, w_gate)
    pos, aux = _route(scores, k)
    tok_ids = jnp.arange(B * T, dtype=jnp.int32)
    inv = _invert(pos.reshape(B * T), tok_ids)
    x2d = x.reshape(B * T, D)
    xc = _gather(x2d, inv, B * k, D, 0)
    y = _ffn(xc, W1, b1, W2, b2)
    out2d = _scatter(y, x2d, inv, D)
    return out2d.reshape(B, T, D), aux[0, 0]
